# Initial kernel scaffold; baseline (speedup 1.0000x reference)
import jax
import jax.numpy as jnp
from jax.experimental import pallas as pl


def kernel(x, table):
    raise NotImplementedError("write your pallas kernel here")


# speedup vs baseline: 3.1178x; 3.1178x over previous
import functools

import jax
import jax.numpy as jnp
from jax import lax
from jax.experimental import pallas as pl
from jax.experimental.pallas import tpu as pltpu
from jax.experimental.pallas import tpu_sc as plsc

NC, NS = 2, 16
NW = NC * NS
CHUNK = 128


@functools.lru_cache(maxsize=None)
def _build_gather(n_rows, d, n_chunks):
    mesh = plsc.VectorSubcoreMesh(core_axis_name="c", subcore_axis_name="s")

    @functools.partial(
        pl.kernel,
        mesh=mesh,
        out_type=jax.ShapeDtypeStruct((n_rows, d), jnp.float32),
        scratch_types=[
            pltpu.VMEM((n_chunks, CHUNK), jnp.int32),
            pltpu.VMEM((CHUNK, d), jnp.float32),
            pltpu.SemaphoreType.DMA,
        ],
        compiler_params=pltpu.CompilerParams(use_tc_tiling_on_sc=False),
    )
    def gather_kernel(table_hbm, idx_hbm, out_hbm, idx_v, buf, sem):
        wid = lax.axis_index("c") * NS + lax.axis_index("s")
        pltpu.sync_copy(idx_hbm.at[wid], idx_v)
        base = wid * (n_chunks * CHUNK)

        def body(g, carry):
            pltpu.async_copy(table_hbm.at[idx_v.at[g]], buf, sem).wait()
            pltpu.sync_copy(buf, out_hbm.at[pl.ds(base + g * CHUNK, CHUNK)])
            return carry

        lax.fori_loop(0, n_chunks, body, 0)

    return gather_kernel


def kernel(x, table):
    b, g, two = x.shape
    d = table.shape[1]
    n_total = b * g * two
    n_chunks = n_total // (NW * CHUNK)
    idx = x.reshape(NW, n_chunks, CHUNK)
    rows = _build_gather(n_total, d, n_chunks)(table, idx)
    return rows.reshape(b, g, two * d)


# speedup vs baseline: 3.5838x; 1.1495x over previous
import functools

import jax
import jax.numpy as jnp
from jax import lax
from jax.experimental import pallas as pl
from jax.experimental.pallas import tpu as pltpu
from jax.experimental.pallas import tpu_sc as plsc

NC, NS = 2, 16
NW = NC * NS
CHUNK = 128
NB = 4


@functools.lru_cache(maxsize=None)
def _build_gather(n_rows, d, n_chunks):
    mesh = plsc.VectorSubcoreMesh(core_axis_name="c", subcore_axis_name="s")

    @functools.partial(
        pl.kernel,
        mesh=mesh,
        out_type=jax.ShapeDtypeStruct((n_rows, d), jnp.float32),
        scratch_types=[
            pltpu.VMEM((n_chunks, CHUNK), jnp.int32),
            pltpu.VMEM((NB, CHUNK, d), jnp.float32),
            pltpu.SemaphoreType.DMA((NB,)),
            pltpu.SemaphoreType.DMA((NB,)),
        ],
        compiler_params=pltpu.CompilerParams(use_tc_tiling_on_sc=False),
    )
    def gather_kernel(table_hbm, idx_hbm, out_hbm, idx_v, bufs, gsem, wsem):
        wid = lax.axis_index("c") * NS + lax.axis_index("s")
        pltpu.sync_copy(idx_hbm.at[wid], idx_v)
        base = wid * (n_chunks * CHUNK)

        def gather_start(i, b):
            pltpu.async_copy(table_hbm.at[idx_v.at[i]], bufs.at[b], gsem.at[b])

        def gather_wait(i, b):
            pltpu.make_async_copy(
                table_hbm.at[idx_v.at[i]], bufs.at[b], gsem.at[b]).wait()

        def write_start(i, b):
            pltpu.async_copy(
                bufs.at[b], out_hbm.at[pl.ds(base + i * CHUNK, CHUNK)],
                wsem.at[b])

        def write_wait(i, b):
            pltpu.make_async_copy(
                bufs.at[b], out_hbm.at[pl.ds(base + i * CHUNK, CHUNK)],
                wsem.at[b]).wait()

        for b in range(NB):
            gather_start(b, b)

        def group(g, carry):
            for b in range(NB):
                i = g * NB + b
                gather_wait(i, b)
                write_start(i, b)
                write_wait(i, b)
                gather_start(i + NB, b)
            return carry

        lax.fori_loop(0, n_chunks // NB - 1, group, 0)

        for b in range(NB):
            i = n_chunks - NB + b
            gather_wait(i, b)
            write_start(i, b)
            write_wait(i, b)

    return gather_kernel


def kernel(x, table):
    b, g, two = x.shape
    d = table.shape[1]
    n_total = b * g * two
    n_chunks = n_total // (NW * CHUNK)
    idx = x.reshape(NW, n_chunks, CHUNK)
    rows = _build_gather(n_total, d, n_chunks)(table, idx)
    return rows.reshape(b, g, two * d)
